# baseline (speedup 1.0000x reference)
import jax
import jax.numpy as jnp
from jax import lax
from jax.experimental import pallas as pl

_E = 16


def _mlp_body(x_ref, skill_ref, w1_ref, b1_ref, w2_ref, b2_ref, w3_ref,
              b3_ref, out_ref):
    x = x_ref[...]
    skill = skill_ref[...]
    zero = jnp.zeros((), jnp.bfloat16)
    h2_sel = jnp.zeros((x.shape[0], w2_ref.shape[2]), jnp.bfloat16)
    for e in range(_E):
        h = jnp.dot(x, w1_ref[e], preferred_element_type=jnp.float32)
        h = jnp.maximum(h + b1_ref[e], 0.0).astype(jnp.bfloat16)
        h = jnp.dot(h, w2_ref[e], preferred_element_type=jnp.float32)
        h = jnp.maximum(h + b2_ref[e], 0.0).astype(jnp.bfloat16)
        h2_sel = h2_sel + jnp.where(skill == e, h, zero)

    onehot = (skill ==
              lax.broadcasted_iota(jnp.int32, (x.shape[0], _E), 1)
              ).astype(jnp.bfloat16)
    w3_sel = jnp.dot(onehot, w3_ref[...], preferred_element_type=jnp.float32)
    b3_sel = jnp.dot(onehot, b3_ref[...], preferred_element_type=jnp.float32)
    logit = jnp.sum(h2_sel.astype(jnp.float32) * w3_sel, axis=1,
                    keepdims=True) + b3_sel
    out_ref[...] = jax.nn.sigmoid(logit)


def kernel(observation, action, skill_idx, W1, b1, W2, b2, W3, b3):
    batch = observation.shape[0]
    in_dim = observation.shape[1] + action.shape[1]
    h1 = W1.shape[2]
    h2 = W2.shape[2]

    x = jnp.concatenate([observation, action], axis=1).astype(jnp.bfloat16)
    skill = skill_idx.astype(jnp.int32).reshape(batch, 1)
    b1r = b1.reshape(_E, 1, h1)
    b2r = b2.reshape(_E, 1, h2)
    w3r = W3.reshape(_E, h2)
    b3r = b3.reshape(_E, 1)

    out = pl.pallas_call(
        _mlp_body,
        in_specs=[
            pl.BlockSpec((batch, in_dim), lambda: (0, 0)),
            pl.BlockSpec((batch, 1), lambda: (0, 0)),
            pl.BlockSpec((_E, in_dim, h1), lambda: (0, 0, 0)),
            pl.BlockSpec((_E, 1, h1), lambda: (0, 0, 0)),
            pl.BlockSpec((_E, h1, h2), lambda: (0, 0, 0)),
            pl.BlockSpec((_E, 1, h2), lambda: (0, 0, 0)),
            pl.BlockSpec((_E, h2), lambda: (0, 0)),
            pl.BlockSpec((_E, 1), lambda: (0, 0)),
        ],
        out_specs=pl.BlockSpec((batch, 1), lambda: (0, 0)),
        out_shape=jax.ShapeDtypeStruct((batch, 1), jnp.float32),
    )(x, skill, W1.astype(jnp.bfloat16), b1r, W2.astype(jnp.bfloat16),
      b2r, w3r, b3r)
    return out


# speedup vs baseline: 1.1960x; 1.1960x over previous
import jax
import jax.numpy as jnp
from jax import lax
from jax.experimental import pallas as pl

_E = 16


def _mlp_body(obs_ref, act_ref, skill_ref, w1_ref, b1_ref, w2_ref, b2_ref,
              w3_ref, b3_ref, out_ref):
    obs = obs_ref[...].astype(jnp.bfloat16)
    act = act_ref[...].astype(jnp.bfloat16)
    skill = skill_ref[...]
    n_obs = obs.shape[1]
    b1 = b1_ref[...]
    b2 = b2_ref[...]
    zero = jnp.zeros((), jnp.bfloat16)
    h2_sel = jnp.zeros((obs.shape[0], w2_ref.shape[2]), jnp.bfloat16)
    for e in range(_E):
        w1 = w1_ref[e].astype(jnp.bfloat16)
        h = (jnp.dot(obs, w1[:n_obs], preferred_element_type=jnp.float32) +
             jnp.dot(act, w1[n_obs:], preferred_element_type=jnp.float32))
        h = jnp.maximum(h + b1[e][None, :], 0.0).astype(jnp.bfloat16)
        h = jnp.dot(h, w2_ref[e].astype(jnp.bfloat16),
                    preferred_element_type=jnp.float32)
        h = jnp.maximum(h + b2[e][None, :], 0.0).astype(jnp.bfloat16)
        h2_sel = h2_sel + jnp.where(skill == e, h, zero)

    onehot = (skill ==
              lax.broadcasted_iota(jnp.int32, (obs.shape[0], _E), 1)
              ).astype(jnp.bfloat16)
    w3_sel = jnp.dot(onehot, w3_ref[...][:, :, 0],
                     preferred_element_type=jnp.float32)
    b3_sel = jnp.dot(onehot, b3_ref[...], preferred_element_type=jnp.float32)
    logit = jnp.sum(h2_sel.astype(jnp.float32) * w3_sel, axis=1,
                    keepdims=True) + b3_sel
    out_ref[...] = jax.nn.sigmoid(logit)


def kernel(observation, action, skill_idx, W1, b1, W2, b2, W3, b3):
    batch = observation.shape[0]
    skill = skill_idx.astype(jnp.int32).reshape(batch, 1)
    out = pl.pallas_call(
        _mlp_body,
        out_shape=jax.ShapeDtypeStruct((batch, 1), jnp.float32),
    )(observation, action, skill, W1, b1, W2, b2, W3, b3)
    return out


# speedup vs baseline: 1.4030x; 1.1731x over previous
import jax
import jax.numpy as jnp
from jax import lax
from jax.experimental import pallas as pl
from jax.experimental.pallas import tpu as pltpu

_E = 16
_B = 1024
_OBS = 256
_ACT = 64
_H = 256


def _lhst_dot(a_t, b, prec=None):
    return lax.dot_general(a_t, b, (((0,), (0,)), ((), ())),
                           preferred_element_type=jnp.float32,
                           precision=prec)


def _mlp_body(obs_hbm, act_hbm, skill_hbm, w1_hbm, b1_hbm, w2_hbm, b2_hbm,
              w3b_hbm, out_hbm, obs_v, act_v, skill_v, b1_v, b2_v, w3b_v,
              w1_v, w2_v, out_v, sem_in, sem_w):
    pro = [
        pltpu.make_async_copy(obs_hbm, obs_v, sem_in),
        pltpu.make_async_copy(act_hbm, act_v, sem_in),
        pltpu.make_async_copy(skill_hbm, skill_v, sem_in),
        pltpu.make_async_copy(b1_hbm, b1_v, sem_in),
        pltpu.make_async_copy(b2_hbm, b2_v, sem_in),
        pltpu.make_async_copy(w3b_hbm, w3b_v, sem_in),
    ]
    for c in pro:
        c.start()
    wc = [[None, None], [None, None]]
    wc[0][0] = pltpu.make_async_copy(w1_hbm.at[0], w1_v.at[0], sem_w.at[0])
    wc[1][0] = pltpu.make_async_copy(w2_hbm.at[0], w2_v.at[0], sem_w.at[0])
    wc[0][0].start()
    wc[1][0].start()
    for c in pro:
        c.wait()

    obs = obs_v[...].astype(jnp.bfloat16)
    act = act_v[...].astype(jnp.bfloat16)
    b1 = b1_v[...]
    b2 = b2_v[...]
    sk8 = skill_v[...]
    skill = jnp.concatenate(
        [jnp.swapaxes(sk8[r:r + 1, :], 0, 1) for r in range(8)], axis=0)

    zero = jnp.zeros((), jnp.bfloat16)
    h2_sel = jnp.zeros((_B, _H), jnp.bfloat16)
    for e in range(_E):
        b = e & 1
        if e + 1 < _E:
            nb = 1 - b
            wc[0][nb] = pltpu.make_async_copy(
                w1_hbm.at[e + 1], w1_v.at[nb], sem_w.at[nb])
            wc[1][nb] = pltpu.make_async_copy(
                w2_hbm.at[e + 1], w2_v.at[nb], sem_w.at[nb])
            wc[0][nb].start()
            wc[1][nb].start()
        wc[0][b].wait()
        wc[1][b].wait()
        w1 = w1_v[b].astype(jnp.bfloat16)
        w2 = w2_v[b].astype(jnp.bfloat16)
        h = _lhst_dot(obs, w1[:_OBS]) + _lhst_dot(act, w1[_OBS:])
        h = jnp.maximum(h + b1[e][None, :], 0.0).astype(jnp.bfloat16)
        h = jnp.dot(h, w2, preferred_element_type=jnp.float32)
        h = jnp.maximum(h + b2[e][None, :], 0.0).astype(jnp.bfloat16)
        h2_sel = h2_sel + jnp.where(skill == e, h, zero)

    onehot = (skill == lax.broadcasted_iota(jnp.int32, (_B, _E), 1)
              ).astype(jnp.bfloat16)
    w3b = w3b_v[...]
    w3_sel = jnp.dot(onehot, w3b[:, :_H], preferred_element_type=jnp.float32)
    b3_sel = jnp.dot(onehot, w3b[:, _H:], preferred_element_type=jnp.float32)
    logit = jnp.sum(h2_sel.astype(jnp.float32) * w3_sel, axis=1,
                    keepdims=True) + b3_sel
    out_v[...] = jax.nn.sigmoid(logit)
    done = pltpu.make_async_copy(out_v, out_hbm, sem_in)
    done.start()
    done.wait()


def kernel(observation, action, skill_idx, W1, b1, W2, b2, W3, b3):
    obs_t = observation.T
    act_t = action.T
    skill = skill_idx.astype(jnp.int32).reshape(8, 128)
    w3b = jnp.concatenate([W3.reshape(_E, _H), b3], axis=1)

    out = pl.pallas_call(
        _mlp_body,
        in_specs=[pl.BlockSpec(memory_space=pl.ANY)] * 8,
        out_specs=pl.BlockSpec(memory_space=pl.ANY),
        out_shape=jax.ShapeDtypeStruct((_B, 1), jnp.float32),
        scratch_shapes=[
            pltpu.VMEM((_OBS, _B), jnp.float32),
            pltpu.VMEM((_ACT, _B), jnp.float32),
            pltpu.VMEM((8, 128), jnp.int32),
            pltpu.VMEM((_E, _H), jnp.float32),
            pltpu.VMEM((_E, _H), jnp.float32),
            pltpu.VMEM((_E, _H + 1), jnp.float32),
            pltpu.VMEM((2, _OBS + _ACT, _H), jnp.float32),
            pltpu.VMEM((2, _H, _H), jnp.float32),
            pltpu.VMEM((_B, 1), jnp.float32),
            pltpu.SemaphoreType.DMA,
            pltpu.SemaphoreType.DMA((2,)),
        ],
    )(obs_t, act_t, skill, W1, b1, W2, b2, w3b)
    return out
